# baseline (speedup 1.0000x reference)
import functools

import jax
import jax.numpy as jnp
from jax import lax
from jax.experimental import pallas as pl
from jax.experimental.pallas import tpu as pltpu
from jax.experimental.pallas import tpu_sc as plsc

N = 10000
D = 128
E = 320000
NC = 2
NS = 16
NW = NC * NS
K = 32
NCHUNK = 320
EP = NW * NCHUNK * K
NBUF = 8
NPH = 8
HCH = NCHUNK // NPH
NP = 10240
RPT = NP // NS

_mesh = plsc.VectorSubcoreMesh(core_axis_name="c", subcore_axis_name="s")


@functools.partial(
    pl.kernel,
    out_type=jax.ShapeDtypeStruct((NC, NP), jnp.float32),
    mesh=_mesh,
    scratch_types=[
        pltpu.VMEM((NCHUNK, K), jnp.int32),
        pltpu.VMEM((K,), jnp.float32),
        pltpu.VMEM_SHARED((NP,), jnp.float32),
    ],
)
def _deg_kernel(col_hbm, zeros_hbm, out_hbm, col_v, ones_v, acc_sp):
    c = lax.axis_index("c")
    s = lax.axis_index("s")
    wid = s * NC + c
    pltpu.sync_copy(zeros_hbm.at[pl.ds(s * (NP // NS), NP // NS)],
                    acc_sp.at[pl.ds(s * (NP // NS), NP // NS)])
    pltpu.sync_copy(col_hbm.at[wid], col_v)
    for i in range(K // 16):
        ones_v[pl.ds(i * 16, 16)] = jnp.ones((16,), jnp.float32)
    plsc.subcore_barrier()

    @pl.loop(0, NCHUNK)
    def _(g):
        pltpu.sync_copy(ones_v, acc_sp.at[col_v.at[g]], add=True)

    plsc.subcore_barrier()
    pltpu.sync_copy(acc_sp.at[pl.ds(s * (NP // NS), NP // NS)],
                    out_hbm.at[c, pl.ds(s * (NP // NS), NP // NS)])


@functools.partial(
    pl.kernel,
    out_type=jax.ShapeDtypeStruct((NC, NP, D), jnp.float32),
    mesh=_mesh,
    scratch_types=[
        pltpu.VMEM((HCH, K), jnp.int32),
        pltpu.VMEM((HCH, K), jnp.int32),
        [pltpu.VMEM((K, D), jnp.float32) for _ in range(NBUF)],
        [pltpu.SemaphoreType.DMA for _ in range(NBUF)],
        [pltpu.SemaphoreType.DMA for _ in range(NBUF)],
        pltpu.VMEM_SHARED((NP, D), jnp.float32),
    ],
)
def _seg_kernel(y_hbm, row_hbm, col_hbm, zeros_hbm, out_hbm,
                row_v, col_v, bufs, gsems, ssems, acc_sp):
    c = lax.axis_index("c")
    s = lax.axis_index("s")
    wid = s * NC + c
    pltpu.sync_copy(zeros_hbm.at[pl.ds(s * RPT, RPT)],
                    acc_sp.at[pl.ds(s * RPT, RPT)])
    plsc.subcore_barrier()

    for ph in range(NPH):
        pltpu.sync_copy(row_hbm.at[wid, pl.ds(ph * HCH, HCH)], row_v)
        pltpu.sync_copy(col_hbm.at[wid, pl.ds(ph * HCH, HCH)], col_v)
        for b in range(NBUF):
            pltpu.async_copy(y_hbm.at[row_v.at[b]], bufs[b], gsems[b])

        @pl.loop(0, HCH, step=NBUF)
        def _(g0):
            for b in range(NBUF):
                g = g0 + b
                pltpu.make_async_copy(y_hbm.at[row_v.at[g]], bufs[b],
                                      gsems[b]).wait()
                pltpu.async_copy(bufs[b], acc_sp.at[col_v.at[g]], ssems[b],
                                 add=True)
            for b in range(NBUF):
                g = g0 + b
                pltpu.make_async_copy(bufs[b], acc_sp.at[col_v.at[g]],
                                      ssems[b]).wait()
                nxt = g0 + b + NBUF

                @pl.when(nxt < HCH)
                def _():
                    pltpu.async_copy(y_hbm.at[row_v.at[nxt]], bufs[b],
                                     gsems[b])

    plsc.subcore_barrier()
    pltpu.sync_copy(acc_sp.at[pl.ds(s * RPT, RPT)],
                    out_hbm.at[c, pl.ds(s * RPT, RPT)])


def _gru_evolve(w, wih, whh, bih, bhh):
    dn = (((1,), (1,)), ((), ()))
    gi = lax.dot_general(w, wih, dn, preferred_element_type=jnp.float32) + bih
    gh = lax.dot_general(w, whh, dn, preferred_element_type=jnp.float32) + bhh
    i_r, i_z, i_n = gi[:, :D], gi[:, D:2 * D], gi[:, 2 * D:]
    h_r, h_z, h_n = gh[:, :D], gh[:, D:2 * D], gh[:, 2 * D:]
    r = lax.logistic(i_r + h_r)
    z = lax.logistic(i_z + h_z)
    n = jnp.tanh(i_n + r * h_n)
    return (1.0 - z) * n + z * w


def _tc_dense_body(x_ref, wlin_ref, blin_ref, deg2_ref,
                   iw0_ref, wih0_ref, whh0_ref, bih0_ref, bhh0_ref,
                   iw1_ref, wih1_ref, whh1_ref, bih1_ref, bhh1_ref,
                   y0_ref, dis_ref, w1_ref):
    deg = deg2_ref[:, 0:1] + deg2_ref[:, 1:2]
    dis = jnp.where(deg > 0.0, lax.rsqrt(jnp.where(deg > 0.0, deg, 1.0)), 0.0)
    dis_ref[...] = dis
    dn = (((1,), (1,)), ((), ()))
    x = lax.dot_general(x_ref[...], wlin_ref[...], dn,
                        preferred_element_type=jnp.float32) + blin_ref[...]
    w0 = _gru_evolve(iw0_ref[...], wih0_ref[...], whh0_ref[...],
                     bih0_ref[...], bhh0_ref[...])
    w1_ref[...] = _gru_evolve(iw1_ref[...], wih1_ref[...], whh1_ref[...],
                              bih1_ref[...], bhh1_ref[...])
    y0_ref[...] = dis * jnp.dot(x, w0, preferred_element_type=jnp.float32)


_tc_dense = pl.pallas_call(
    _tc_dense_body,
    out_shape=(
        jax.ShapeDtypeStruct((N, D), jnp.float32),
        jax.ShapeDtypeStruct((N, 1), jnp.float32),
        jax.ShapeDtypeStruct((D, D), jnp.float32),
    ),
)


def _tc_mid_body(z_ref, dis_ref, w1_ref, y1_ref):
    dis = dis_ref[...]
    x1 = dis * (z_ref[0, :N, :] + z_ref[1, :N, :])
    y1_ref[...] = dis * jnp.dot(x1, w1_ref[...], preferred_element_type=jnp.float32)


_tc_mid = pl.pallas_call(
    _tc_mid_body,
    out_shape=jax.ShapeDtypeStruct((N, D), jnp.float32),
)


def _tc_out_body(z_ref, dis_ref, o_ref):
    o_ref[...] = dis_ref[...] * (z_ref[0, :N, :] + z_ref[1, :N, :])


_tc_out = pl.pallas_call(
    _tc_out_body,
    out_shape=jax.ShapeDtypeStruct((N, D), jnp.float32),
)


def kernel(X, edge_index, W_lin, b_lin,
           init_w0, gru_wih0, gru_whh0, gru_bih0, gru_bhh0,
           init_w1, gru_wih1, gru_whh1, gru_bih1, gru_bhh1):
    ppw = (EP - E) // NW
    epw = E // NW
    padrow = jnp.broadcast_to(41 * lax.iota(jnp.int32, ppw), (NW, ppw))
    padcol = jnp.broadcast_to(N + lax.iota(jnp.int32, ppw), (NW, ppw))
    row = jnp.concatenate([edge_index[0].reshape(NW, epw), padrow], axis=1)
    col = jnp.concatenate([edge_index[1].reshape(NW, epw), padcol], axis=1)
    row = row.reshape(NW, NCHUNK, K)
    col = col.reshape(NW, NCHUNK, K)
    zeros_np = jnp.zeros((NP,), jnp.float32)
    zeros_nd = jnp.zeros((NP, D), jnp.float32)

    deg_p = _deg_kernel(col, zeros_np)
    deg2 = deg_p[:, :N].T

    y0, dis, W1 = _tc_dense(X, W_lin, b_lin.reshape(1, D), deg2,
                            init_w0, gru_wih0, gru_whh0,
                            gru_bih0.reshape(1, 3 * D), gru_bhh0.reshape(1, 3 * D),
                            init_w1, gru_wih1, gru_whh1,
                            gru_bih1.reshape(1, 3 * D), gru_bhh1.reshape(1, 3 * D))

    z0_p = _seg_kernel(y0, row, col, zeros_nd)
    y1 = _tc_mid(z0_p, dis, W1)
    z1_p = _seg_kernel(y1, row, col, zeros_nd)
    return _tc_out(z1_p, dis)


# speedup vs baseline: 1.0862x; 1.0862x over previous
import functools

import jax
import jax.numpy as jnp
from jax import lax
from jax.experimental import pallas as pl
from jax.experimental.pallas import tpu as pltpu
from jax.experimental.pallas import tpu_sc as plsc

N = 10000
D = 128
E = 320000
NC = 2
NS = 16
NW = NC * NS
K = 64
NCHUNK = 160
EP = NW * NCHUNK * K
NBUF = 4
NPH = 4
HCH = NCHUNK // NPH
NP = 10240
RPT = NP // NS

_mesh = plsc.VectorSubcoreMesh(core_axis_name="c", subcore_axis_name="s")


@functools.partial(
    pl.kernel,
    out_type=jax.ShapeDtypeStruct((NC, NP), jnp.float32),
    mesh=_mesh,
    scratch_types=[
        pltpu.VMEM((NCHUNK, K), jnp.int32),
        pltpu.VMEM((K,), jnp.float32),
        pltpu.VMEM_SHARED((NP,), jnp.float32),
    ],
)
def _deg_kernel(col_hbm, zeros_hbm, out_hbm, col_v, ones_v, acc_sp):
    c = lax.axis_index("c")
    s = lax.axis_index("s")
    wid = s * NC + c
    pltpu.sync_copy(zeros_hbm.at[pl.ds(s * (NP // NS), NP // NS)],
                    acc_sp.at[pl.ds(s * (NP // NS), NP // NS)])
    pltpu.sync_copy(col_hbm.at[wid], col_v)
    for i in range(K // 16):
        ones_v[pl.ds(i * 16, 16)] = jnp.ones((16,), jnp.float32)
    plsc.subcore_barrier()

    @pl.loop(0, NCHUNK)
    def _(g):
        pltpu.sync_copy(ones_v, acc_sp.at[col_v.at[g]], add=True)

    plsc.subcore_barrier()
    pltpu.sync_copy(acc_sp.at[pl.ds(s * (NP // NS), NP // NS)],
                    out_hbm.at[c, pl.ds(s * (NP // NS), NP // NS)])


@functools.partial(
    pl.kernel,
    out_type=jax.ShapeDtypeStruct((NC, NP, D), jnp.float32),
    mesh=_mesh,
    scratch_types=[
        pltpu.VMEM((HCH, K), jnp.int32),
        pltpu.VMEM((HCH, K), jnp.int32),
        [pltpu.VMEM((K, D), jnp.float32) for _ in range(NBUF)],
        [pltpu.SemaphoreType.DMA for _ in range(NBUF)],
        [pltpu.SemaphoreType.DMA for _ in range(NBUF)],
        pltpu.VMEM_SHARED((NP, D), jnp.float32),
    ],
    compiler_params=pltpu.CompilerParams(use_tc_tiling_on_sc=False),
)
def _seg_kernel(y_hbm, row_hbm, col_hbm, zeros_hbm, out_hbm,
                row_v, col_v, bufs, gsems, ssems, acc_sp):
    c = lax.axis_index("c")
    s = lax.axis_index("s")
    wid = s * NC + c
    pltpu.sync_copy(zeros_hbm.at[pl.ds(s * RPT, RPT)],
                    acc_sp.at[pl.ds(s * RPT, RPT)])
    plsc.subcore_barrier()

    for ph in range(NPH):
        pltpu.sync_copy(row_hbm.at[wid, pl.ds(ph * HCH, HCH)], row_v)
        pltpu.sync_copy(col_hbm.at[wid, pl.ds(ph * HCH, HCH)], col_v)
        for b in range(NBUF):
            pltpu.async_copy(y_hbm.at[row_v.at[b]], bufs[b], gsems[b])

        @pl.loop(0, HCH, step=NBUF)
        def _(g0):
            for b in range(NBUF):
                g = g0 + b
                pltpu.make_async_copy(y_hbm.at[row_v.at[g]], bufs[b],
                                      gsems[b]).wait()
                pltpu.async_copy(bufs[b], acc_sp.at[col_v.at[g]], ssems[b],
                                 add=True)
            for b in range(NBUF):
                g = g0 + b
                pltpu.make_async_copy(bufs[b], acc_sp.at[col_v.at[g]],
                                      ssems[b]).wait()
                nxt = g0 + b + NBUF

                @pl.when(nxt < HCH)
                def _():
                    pltpu.async_copy(y_hbm.at[row_v.at[nxt]], bufs[b],
                                     gsems[b])

    plsc.subcore_barrier()
    pltpu.sync_copy(acc_sp.at[pl.ds(s * RPT, RPT)],
                    out_hbm.at[c, pl.ds(s * RPT, RPT)])


def _gru_evolve(w, wih, whh, bih, bhh):
    dn = (((1,), (1,)), ((), ()))
    gi = lax.dot_general(w, wih, dn, preferred_element_type=jnp.float32) + bih
    gh = lax.dot_general(w, whh, dn, preferred_element_type=jnp.float32) + bhh
    i_r, i_z, i_n = gi[:, :D], gi[:, D:2 * D], gi[:, 2 * D:]
    h_r, h_z, h_n = gh[:, :D], gh[:, D:2 * D], gh[:, 2 * D:]
    r = lax.logistic(i_r + h_r)
    z = lax.logistic(i_z + h_z)
    n = jnp.tanh(i_n + r * h_n)
    return (1.0 - z) * n + z * w


def _tc_dense_body(x_ref, wlin_ref, blin_ref, deg2_ref,
                   iw0_ref, wih0_ref, whh0_ref, bih0_ref, bhh0_ref,
                   iw1_ref, wih1_ref, whh1_ref, bih1_ref, bhh1_ref,
                   y0_ref, dis_ref, w1_ref):
    deg = deg2_ref[:, 0:1] + deg2_ref[:, 1:2]
    dis = jnp.where(deg > 0.0, lax.rsqrt(jnp.where(deg > 0.0, deg, 1.0)), 0.0)
    dis_ref[...] = dis
    dn = (((1,), (1,)), ((), ()))
    x = lax.dot_general(x_ref[...], wlin_ref[...], dn,
                        preferred_element_type=jnp.float32) + blin_ref[...]
    w0 = _gru_evolve(iw0_ref[...], wih0_ref[...], whh0_ref[...],
                     bih0_ref[...], bhh0_ref[...])
    w1_ref[...] = _gru_evolve(iw1_ref[...], wih1_ref[...], whh1_ref[...],
                              bih1_ref[...], bhh1_ref[...])
    y0_ref[...] = dis * jnp.dot(x, w0, preferred_element_type=jnp.float32)


_tc_dense = pl.pallas_call(
    _tc_dense_body,
    out_shape=(
        jax.ShapeDtypeStruct((N, D), jnp.float32),
        jax.ShapeDtypeStruct((N, 1), jnp.float32),
        jax.ShapeDtypeStruct((D, D), jnp.float32),
    ),
)


def _tc_mid_body(z_ref, dis_ref, w1_ref, y1_ref):
    dis = dis_ref[...]
    x1 = dis * (z_ref[0, :N, :] + z_ref[1, :N, :])
    y1_ref[...] = dis * jnp.dot(x1, w1_ref[...], preferred_element_type=jnp.float32)


_tc_mid = pl.pallas_call(
    _tc_mid_body,
    out_shape=jax.ShapeDtypeStruct((N, D), jnp.float32),
)


def _tc_out_body(z_ref, dis_ref, o_ref):
    o_ref[...] = dis_ref[...] * (z_ref[0, :N, :] + z_ref[1, :N, :])


_tc_out = pl.pallas_call(
    _tc_out_body,
    out_shape=jax.ShapeDtypeStruct((N, D), jnp.float32),
)


def kernel(X, edge_index, W_lin, b_lin,
           init_w0, gru_wih0, gru_whh0, gru_bih0, gru_bhh0,
           init_w1, gru_wih1, gru_whh1, gru_bih1, gru_bhh1):
    ppw = (EP - E) // NW
    epw = E // NW
    padrow = jnp.broadcast_to(41 * lax.iota(jnp.int32, ppw), (NW, ppw))
    padcol = jnp.broadcast_to(N + lax.iota(jnp.int32, ppw), (NW, ppw))
    row = jnp.concatenate([edge_index[0].reshape(NW, epw), padrow], axis=1)
    col = jnp.concatenate([edge_index[1].reshape(NW, epw), padcol], axis=1)
    row = row.reshape(NW, NCHUNK, K)
    col = col.reshape(NW, NCHUNK, K)
    zeros_np = jnp.zeros((NP,), jnp.float32)
    zeros_nd = jnp.zeros((NP, D), jnp.float32)

    deg_p = _deg_kernel(col, zeros_np)
    deg2 = deg_p[:, :N].T

    y0, dis, W1 = _tc_dense(X, W_lin, b_lin.reshape(1, D), deg2,
                            init_w0, gru_wih0, gru_whh0,
                            gru_bih0.reshape(1, 3 * D), gru_bhh0.reshape(1, 3 * D),
                            init_w1, gru_wih1, gru_whh1,
                            gru_bih1.reshape(1, 3 * D), gru_bhh1.reshape(1, 3 * D))

    z0_p = _seg_kernel(y0, row, col, zeros_nd)
    y1 = _tc_mid(z0_p, dis, W1)
    z1_p = _seg_kernel(y1, row, col, zeros_nd)
    return _tc_out(z1_p, dis)


# speedup vs baseline: 1.1136x; 1.0252x over previous
import functools

import jax
import jax.numpy as jnp
from jax import lax
from jax.experimental import pallas as pl
from jax.experimental.pallas import tpu as pltpu
from jax.experimental.pallas import tpu_sc as plsc

N = 10000
D = 128
E = 320000
NC = 2
NS = 16
NW = NC * NS
K = 64
NCHUNK = 160
EP = NW * NCHUNK * K
NBUF = 4
NPH = 2
HCH = NCHUNK // NPH
NP = 10240
RPT = NP // NS

_mesh = plsc.VectorSubcoreMesh(core_axis_name="c", subcore_axis_name="s")


@functools.partial(
    pl.kernel,
    out_type=jax.ShapeDtypeStruct((NC, NP), jnp.float32),
    mesh=_mesh,
    scratch_types=[
        pltpu.VMEM((NCHUNK, K), jnp.int32),
        pltpu.VMEM((K,), jnp.float32),
        pltpu.VMEM_SHARED((NP,), jnp.float32),
    ],
)
def _deg_kernel(col_hbm, zeros_hbm, out_hbm, col_v, ones_v, acc_sp):
    c = lax.axis_index("c")
    s = lax.axis_index("s")
    wid = s * NC + c
    pltpu.sync_copy(zeros_hbm.at[pl.ds(s * (NP // NS), NP // NS)],
                    acc_sp.at[pl.ds(s * (NP // NS), NP // NS)])
    pltpu.sync_copy(col_hbm.at[wid], col_v)
    for i in range(K // 16):
        ones_v[pl.ds(i * 16, 16)] = jnp.ones((16,), jnp.float32)
    plsc.subcore_barrier()

    @pl.loop(0, NCHUNK)
    def _(g):
        pltpu.sync_copy(ones_v, acc_sp.at[col_v.at[g]], add=True)

    plsc.subcore_barrier()
    pltpu.sync_copy(acc_sp.at[pl.ds(s * (NP // NS), NP // NS)],
                    out_hbm.at[c, pl.ds(s * (NP // NS), NP // NS)])


@functools.partial(
    pl.kernel,
    out_type=jax.ShapeDtypeStruct((NC, NP, D), jnp.float32),
    mesh=_mesh,
    scratch_types=[
        pltpu.VMEM((HCH, K), jnp.int32),
        pltpu.VMEM((HCH, K), jnp.int32),
        [pltpu.VMEM((K, D), jnp.float32) for _ in range(NBUF)],
        [pltpu.SemaphoreType.DMA for _ in range(NBUF)],
        [pltpu.SemaphoreType.DMA for _ in range(NBUF)],
        pltpu.VMEM_SHARED((NP, D), jnp.float32),
    ],
    compiler_params=pltpu.CompilerParams(use_tc_tiling_on_sc=False),
)
def _seg_kernel(y_hbm, row_hbm, col_hbm, zeros_hbm, out_hbm,
                row_v, col_v, bufs, gsems, ssems, acc_sp):
    c = lax.axis_index("c")
    s = lax.axis_index("s")
    wid = s * NC + c
    pltpu.sync_copy(zeros_hbm.at[pl.ds(s * RPT, RPT)],
                    acc_sp.at[pl.ds(s * RPT, RPT)])
    plsc.subcore_barrier()

    for ph in range(NPH):
        pltpu.sync_copy(row_hbm.at[wid, pl.ds(ph * HCH, HCH)], row_v)
        pltpu.sync_copy(col_hbm.at[wid, pl.ds(ph * HCH, HCH)], col_v)
        for b in range(NBUF):
            pltpu.async_copy(y_hbm.at[row_v.at[b]], bufs[b], gsems[b])

        @pl.loop(0, HCH, step=NBUF)
        def _(g0):
            for b in range(NBUF):
                g = g0 + b
                pltpu.make_async_copy(y_hbm.at[row_v.at[g]], bufs[b],
                                      gsems[b]).wait()
                pltpu.async_copy(bufs[b], acc_sp.at[col_v.at[g]], ssems[b],
                                 add=True)
            for b in range(NBUF):
                g = g0 + b
                pltpu.make_async_copy(bufs[b], acc_sp.at[col_v.at[g]],
                                      ssems[b]).wait()
                nxt = g0 + b + NBUF

                @pl.when(nxt < HCH)
                def _():
                    pltpu.async_copy(y_hbm.at[row_v.at[nxt]], bufs[b],
                                     gsems[b])

    plsc.subcore_barrier()
    pltpu.sync_copy(acc_sp.at[pl.ds(s * RPT, RPT)],
                    out_hbm.at[c, pl.ds(s * RPT, RPT)])


def _gru_evolve(w, wih, whh, bih, bhh):
    dn = (((1,), (1,)), ((), ()))
    gi = lax.dot_general(w, wih, dn, preferred_element_type=jnp.float32) + bih
    gh = lax.dot_general(w, whh, dn, preferred_element_type=jnp.float32) + bhh
    i_r, i_z, i_n = gi[:, :D], gi[:, D:2 * D], gi[:, 2 * D:]
    h_r, h_z, h_n = gh[:, :D], gh[:, D:2 * D], gh[:, 2 * D:]
    r = lax.logistic(i_r + h_r)
    z = lax.logistic(i_z + h_z)
    n = jnp.tanh(i_n + r * h_n)
    return (1.0 - z) * n + z * w


def _tc_dense_body(x_ref, wlin_ref, blin_ref, deg2_ref,
                   iw0_ref, wih0_ref, whh0_ref, bih0_ref, bhh0_ref,
                   iw1_ref, wih1_ref, whh1_ref, bih1_ref, bhh1_ref,
                   y0_ref, dis_ref, w1_ref):
    deg = deg2_ref[:, 0:1] + deg2_ref[:, 1:2]
    dis = jnp.where(deg > 0.0, lax.rsqrt(jnp.where(deg > 0.0, deg, 1.0)), 0.0)
    dis_ref[...] = dis
    dn = (((1,), (1,)), ((), ()))
    x = lax.dot_general(x_ref[...], wlin_ref[...], dn,
                        preferred_element_type=jnp.float32) + blin_ref[...]
    w0 = _gru_evolve(iw0_ref[...], wih0_ref[...], whh0_ref[...],
                     bih0_ref[...], bhh0_ref[...])
    w1_ref[...] = _gru_evolve(iw1_ref[...], wih1_ref[...], whh1_ref[...],
                              bih1_ref[...], bhh1_ref[...])
    y0_ref[...] = dis * jnp.dot(x, w0, preferred_element_type=jnp.float32)


_tc_dense = pl.pallas_call(
    _tc_dense_body,
    out_shape=(
        jax.ShapeDtypeStruct((N, D), jnp.float32),
        jax.ShapeDtypeStruct((N, 1), jnp.float32),
        jax.ShapeDtypeStruct((D, D), jnp.float32),
    ),
)


def _tc_mid_body(z_ref, dis_ref, w1_ref, y1_ref):
    dis = dis_ref[...]
    x1 = dis * (z_ref[0, :N, :] + z_ref[1, :N, :])
    y1_ref[...] = dis * jnp.dot(x1, w1_ref[...], preferred_element_type=jnp.float32)


_tc_mid = pl.pallas_call(
    _tc_mid_body,
    out_shape=jax.ShapeDtypeStruct((N, D), jnp.float32),
)


def _tc_out_body(z_ref, dis_ref, o_ref):
    o_ref[...] = dis_ref[...] * (z_ref[0, :N, :] + z_ref[1, :N, :])


_tc_out = pl.pallas_call(
    _tc_out_body,
    out_shape=jax.ShapeDtypeStruct((N, D), jnp.float32),
)


def kernel(X, edge_index, W_lin, b_lin,
           init_w0, gru_wih0, gru_whh0, gru_bih0, gru_bhh0,
           init_w1, gru_wih1, gru_whh1, gru_bih1, gru_bhh1):
    ppw = (EP - E) // NW
    epw = E // NW
    padrow = jnp.broadcast_to(41 * lax.iota(jnp.int32, ppw), (NW, ppw))
    padcol = jnp.broadcast_to(N + lax.iota(jnp.int32, ppw), (NW, ppw))
    row = jnp.concatenate([edge_index[0].reshape(NW, epw), padrow], axis=1)
    col = jnp.concatenate([edge_index[1].reshape(NW, epw), padcol], axis=1)
    row = row.reshape(NW, NCHUNK, K)
    col = col.reshape(NW, NCHUNK, K)
    zeros_np = jnp.zeros((NP,), jnp.float32)
    zeros_nd = jnp.zeros((NP, D), jnp.float32)

    deg_p = _deg_kernel(col, zeros_np)
    deg2 = deg_p[:, :N].T

    y0, dis, W1 = _tc_dense(X, W_lin, b_lin.reshape(1, D), deg2,
                            init_w0, gru_wih0, gru_whh0,
                            gru_bih0.reshape(1, 3 * D), gru_bhh0.reshape(1, 3 * D),
                            init_w1, gru_wih1, gru_whh1,
                            gru_bih1.reshape(1, 3 * D), gru_bhh1.reshape(1, 3 * D))

    z0_p = _seg_kernel(y0, row, col, zeros_nd)
    y1 = _tc_mid(z0_p, dis, W1)
    z1_p = _seg_kernel(y1, row, col, zeros_nd)
    return _tc_out(z1_p, dis)
